# baseline (speedup 1.0000x reference)
import functools

import jax
import jax.numpy as jnp
from jax import lax
from jax.experimental import pallas as pl
from jax.experimental.pallas import tpu as pltpu
from jax.experimental.pallas import tpu_sc as plsc

_PCT = 95
_MIN_SCALE = 1e-06
_MAX_SCALE = 1000000.0
_INF_BITS = 0x7F800000

_N = 128 * 8192
_NS = 16
_L = 16
_PER_T = _N // _NS
_VECS = _PER_T // _L


def _tc1_body(x_ref, p_ref, s_ref):
    x = x_ref[:]
    a = jnp.abs(x)
    mask = a > 1e-08
    n0 = jnp.sum(mask.astype(jnp.int32))
    n0f = n0.astype(jnp.float32)
    s = jnp.sum(jnp.where(mask, a, 0.0))
    mean = s / jnp.maximum(n0f, 1.0)
    d = a - mean
    ss = jnp.sum(jnp.where(mask, d * d, 0.0))
    var = ss / jnp.maximum(n0f - 1.0, 1.0)
    std = jnp.sqrt(var)
    refined = mask & (jnp.abs(d) <= 3.0 * std)
    nr = jnp.sum(refined.astype(jnp.int32))
    use_refined = (n0 > 10) & (nr > 0)
    n = jnp.where(use_refined, nr, n0)
    k = jnp.clip((_PCT * n) // 100, 0, n - 1)
    r = k + 1
    sel = (refined & use_refined) | (mask & jnp.logical_not(use_refined))
    bits = lax.bitcast_convert_type(a, jnp.int32)
    p_ref[:] = jnp.where(sel, bits, _INF_BITS)
    rows = lax.broadcasted_iota(jnp.int32, (8, 128), 0)
    s_ref[:] = jnp.where(
        rows == 0, r, jnp.where(rows == 1, n, jnp.where(rows == 2, n0, 0))
    )


_PER_W = _N // (2 * _NS)


def _cum_search(ghist_v, tmpa_v, tmpb_v, r_spl, nb):
    def chunk(j, carry):
        tot, bfound, cbel = carry
        h = ghist_v[pl.ds(j * _L, _L)]
        cs = plsc.cumsum(h) + tot
        ge = cs >= r_spl
        anyv = plsc.all_reduce_population_count(ge)
        ffs = plsc.all_reduce_ffs(ge)
        ffs = jnp.minimum(ffs, _L - 1)
        excl = cs - h
        tmpa_v[...] = excl
        gathered = plsc.load_gather(tmpa_v, [ffs])
        tmpb_v[...] = cs
        tot_new = plsc.load_gather(tmpb_v, [jnp.full((_L,), _L - 1, jnp.int32)])
        newly = (bfound < 0) & (anyv > 0)
        bfound = jnp.where(newly, j * _L + ffs, bfound)
        cbel = jnp.where(newly, gathered, cbel)
        return (tot_new, bfound, cbel)

    zero = jnp.zeros((_L,), jnp.int32)
    init = (zero, zero - 1, zero)
    tot, bfound, cbel = lax.fori_loop(0, nb // _L, chunk, init)
    return jnp.maximum(bfound, 0), cbel


_SKEW = 2081
_ROW = 2080
_DUMMY = 2064


def _zero_hist(h_ref, nb):
    zero = jnp.zeros((_L,), jnp.int32)
    for row in range(_NS):

        @plsc.parallel_loop(0, nb // _L, unroll=8)
        def _(col, row=row):
            h_ref[pl.ds(row * _SKEW + col * _L, _L)] = zero


def _hist_round(p_v, h_ref, rowbuf_v, sh_ref, ghist_v, tmpa_v, tmpb_v, sid,
                r_spl, nb, bin_fn, mask_fn):
    _zero_hist(h_ref, nb)
    lane_off = lax.broadcasted_iota(jnp.int32, (_L,), 0) * _SKEW
    ones = jnp.ones((_L,), jnp.int32)

    @plsc.parallel_loop(0, _VECS, unroll=16)
    def _(i):
        v = p_v[pl.ds(i * _L, _L)]
        bins = jnp.where(mask_fn(v), bin_fn(v), _DUMMY)
        plsc.addupdate_scatter(h_ref, [lane_off + bins], ones)

    @plsc.parallel_loop(0, nb // _L, unroll=4)
    def _(j):
        acc = jnp.zeros((_L,), jnp.int32)
        for row in range(_NS):
            acc = acc + h_ref[pl.ds(row * _SKEW + j * _L, _L)]
        rowbuf_v[pl.ds(j * _L, _L)] = acc

    pltpu.sync_copy(rowbuf_v.at[pl.ds(0, nb)], sh_ref.at[pl.ds(sid * nb, nb)])
    plsc.subcore_barrier()
    for row in range(_NS):
        pltpu.sync_copy(
            sh_ref.at[pl.ds(row * nb, nb)], h_ref.at[pl.ds(row * _ROW, nb)]
        )

    @plsc.parallel_loop(0, nb // _L, unroll=4)
    def _(j):
        acc = jnp.zeros((_L,), jnp.int32)
        for row in range(_NS):
            acc = acc + h_ref[pl.ds(row * _ROW + j * _L, _L)]
        ghist_v[pl.ds(j * _L, _L)] = acc

    return _cum_search(ghist_v, tmpa_v, tmpb_v, r_spl, nb)


def _sc_select_make():
    mesh = plsc.VectorSubcoreMesh(
        core_axis_name="c", subcore_axis_name="s", num_cores=2, num_subcores=_NS
    )

    @functools.partial(
        pl.kernel,
        out_type=jax.ShapeDtypeStruct((_N,), jnp.int32),
        mesh=mesh,
        compiler_params=pltpu.CompilerParams(needs_layout_passes=False),
        scratch_types=dict(
            p_v=pltpu.VMEM((_PER_T,), jnp.int32),
            h_v=pltpu.VMEM((_NS * _ROW,), jnp.int32),
            rowbuf_v=pltpu.VMEM((2048,), jnp.int32),
            ghist_v=pltpu.VMEM((2048,), jnp.int32),
            r_v=pltpu.VMEM((3 * _L,), jnp.int32),
            tmpa_v=pltpu.VMEM((_L,), jnp.int32),
            tmpb_v=pltpu.VMEM((_L,), jnp.int32),
            sh_a=pltpu.VMEM_SHARED((_NS * 1024,), jnp.int32),
            sh_b=pltpu.VMEM_SHARED((_NS * 2048,), jnp.int32),
            sh_c=pltpu.VMEM_SHARED((_NS * 1024,), jnp.int32),
        ),
    )
    def sc_select(p_hbm, x_hbm, st_hbm, o_hbm, *, p_v, h_v, rowbuf_v, ghist_v,
                  r_v, tmpa_v, tmpb_v, sh_a, sh_b, sh_c):
        cid = lax.axis_index("c")
        sid = lax.axis_index("s")
        pltpu.sync_copy(p_hbm.at[pl.ds(sid * _PER_T, _PER_T)], p_v)
        pltpu.sync_copy(st_hbm, r_v)
        r1 = r_v[pl.ds(0, _L)]

        b1, cb1 = _hist_round(
            p_v, h_v, rowbuf_v, sh_a, ghist_v, tmpa_v, tmpb_v, sid, r1, 1024,
            lambda v: lax.shift_right_logical(v, 21),
            lambda v: jnp.ones((_L,), jnp.bool_),
        )
        r2 = r1 - cb1

        b2, cb2 = _hist_round(
            p_v, h_v, rowbuf_v, sh_b, ghist_v, tmpa_v, tmpb_v, sid, r2, 2048,
            lambda v: lax.shift_right_logical(v, 10) & 0x7FF,
            lambda v: lax.shift_right_logical(v, 21) == b1,
        )
        r3 = r2 - cb2
        pre2 = (b1 << 11) | b2

        b3, _ = _hist_round(
            p_v, h_v, rowbuf_v, sh_c, ghist_v, tmpa_v, tmpb_v, sid, r3, 1024,
            lambda v: v & 0x3FF,
            lambda v: lax.shift_right_logical(v, 10) == pre2,
        )

        ans = (b1 << 21) | (b2 << 10) | b3

        n_spl = r_v[pl.ds(_L, _L)]
        n0_spl = r_v[pl.ds(2 * _L, _L)]
        one = jnp.full((_L,), 1.0, jnp.float32)
        val = plsc.bitcast(ans, jnp.float32)
        val = jnp.where(n_spl == 0, one, val)
        val = jnp.clip(val, _MIN_SCALE, _MAX_SCALE)
        val = jnp.where(n0_spl == 0, one, val)
        val = jnp.clip(val, _MIN_SCALE, _MAX_SCALE)
        recip = one / (val + 1e-08)

        wid = sid * 2 + cid
        base = wid * _PER_W
        xbuf = p_v.at[pl.ds(0, _PER_W)]
        pltpu.sync_copy(x_hbm.at[pl.ds(base, _PER_W)], xbuf)

        @plsc.parallel_loop(0, _PER_W // _L, unroll=16)
        def _(i):
            xv = plsc.bitcast(p_v[pl.ds(i * _L, _L)], jnp.float32)
            p_v[pl.ds(i * _L, _L)] = plsc.bitcast(xv * recip, jnp.int32)

        pltpu.sync_copy(xbuf, o_hbm.at[pl.ds(base, _PER_W)])

    return sc_select


def kernel(x):
    p, stats = pl.pallas_call(
        _tc1_body,
        out_shape=(
            jax.ShapeDtypeStruct(x.shape, jnp.int32),
            jax.ShapeDtypeStruct((8, 128), jnp.int32),
        ),
    )(x)
    stvec = jnp.concatenate(
        [
            jnp.broadcast_to(stats[0, 0], (_L,)),
            jnp.broadcast_to(stats[1, 0], (_L,)),
            jnp.broadcast_to(stats[2, 0], (_L,)),
        ]
    )
    x_i = lax.bitcast_convert_type(x, jnp.int32).reshape(-1)
    o_i = _sc_select_make()(p.reshape(-1), x_i, stvec)
    return lax.bitcast_convert_type(o_i, jnp.float32).reshape(x.shape)


# speedup vs baseline: 1.1824x; 1.1824x over previous
import functools

import jax
import jax.numpy as jnp
from jax import lax
from jax.experimental import pallas as pl
from jax.experimental.pallas import tpu as pltpu
from jax.experimental.pallas import tpu_sc as plsc

_PCT = 95
_MIN_SCALE = 1e-06
_MAX_SCALE = 1000000.0
_INF_BITS = 0x7F800000

_N = 128 * 8192
_NS = 16
_L = 16
_PER_T = _N // _NS
_VECS = _PER_T // _L


def _tc1_body(x_ref, p_ref, s_ref):
    x = x_ref[:]
    a = jnp.abs(x)
    mask = a > 1e-08
    n0 = jnp.sum(mask.astype(jnp.int32))
    n0f = n0.astype(jnp.float32)
    s = jnp.sum(jnp.where(mask, a, 0.0))
    mean = s / jnp.maximum(n0f, 1.0)
    d = a - mean
    ss = jnp.sum(jnp.where(mask, d * d, 0.0))
    var = ss / jnp.maximum(n0f - 1.0, 1.0)
    std = jnp.sqrt(var)
    refined = mask & (jnp.abs(d) <= 3.0 * std)
    nr = jnp.sum(refined.astype(jnp.int32))
    use_refined = (n0 > 10) & (nr > 0)
    n = jnp.where(use_refined, nr, n0)
    k = jnp.clip((_PCT * n) // 100, 0, n - 1)
    r = k + 1
    sel = (refined & use_refined) | (mask & jnp.logical_not(use_refined))
    bits = lax.bitcast_convert_type(a, jnp.int32)
    p_ref[:] = jnp.where(sel, bits, _INF_BITS)
    cols = lax.broadcasted_iota(jnp.int32, (8, 128), 1)
    s_ref[:] = jnp.where(
        cols < 16, r, jnp.where(cols < 32, n, jnp.where(cols < 48, n0, 0))
    )


def _tc2_body(x_ref, a_ref, s_ref, o_ref):
    ans = a_ref[0]
    n = s_ref[16]
    n0 = s_ref[32]
    val = lax.bitcast_convert_type(ans, jnp.float32)
    val = jnp.where(n == 0, 1.0, val)
    value = jnp.clip(val, _MIN_SCALE, _MAX_SCALE)
    value = jnp.where(n0 == 0, 1.0, value)
    value = jnp.clip(value, _MIN_SCALE, _MAX_SCALE)
    o_ref[:] = x_ref[:] / (value + 1e-08)


def _cum_search(ghist_v, tmpa_v, tmpb_v, r_spl, nb):
    def chunk(j, carry):
        tot, bfound, cbel = carry
        h = ghist_v[pl.ds(j * _L, _L)]
        cs = plsc.cumsum(h) + tot
        ge = cs >= r_spl
        anyv = plsc.all_reduce_population_count(ge)
        ffs = plsc.all_reduce_ffs(ge)
        ffs = jnp.minimum(ffs, _L - 1)
        excl = cs - h
        tmpa_v[...] = excl
        gathered = plsc.load_gather(tmpa_v, [ffs])
        tmpb_v[...] = cs
        tot_new = plsc.load_gather(tmpb_v, [jnp.full((_L,), _L - 1, jnp.int32)])
        newly = (bfound < 0) & (anyv > 0)
        bfound = jnp.where(newly, j * _L + ffs, bfound)
        cbel = jnp.where(newly, gathered, cbel)
        return (tot_new, bfound, cbel)

    zero = jnp.zeros((_L,), jnp.int32)
    init = (zero, zero - 1, zero)
    tot, bfound, cbel = lax.fori_loop(0, nb // _L, chunk, init)
    return jnp.maximum(bfound, 0), cbel


_SKEW = 2081
_ROW = 2080
_DUMMY = 2064


def _zero_hist(h_ref, nb):
    zero = jnp.zeros((_L,), jnp.int32)
    for row in range(_NS):

        @plsc.parallel_loop(0, nb // _L, unroll=8)
        def _(col, row=row):
            h_ref[pl.ds(row * _SKEW + col * _L, _L)] = zero


def _hist_round(p_v, h_ref, rowbuf_v, sh_ref, ghist_v, tmpa_v, tmpb_v, sid,
                r_spl, nb, bin_fn, mask_fn):
    _zero_hist(h_ref, nb)
    lane_off = lax.broadcasted_iota(jnp.int32, (_L,), 0) * _SKEW
    ones = jnp.ones((_L,), jnp.int32)

    @plsc.parallel_loop(0, _VECS, unroll=16)
    def _(i):
        v = p_v[pl.ds(i * _L, _L)]
        bins = jnp.where(mask_fn(v), bin_fn(v), _DUMMY)
        plsc.addupdate_scatter(h_ref, [lane_off + bins], ones)

    @plsc.parallel_loop(0, nb // _L, unroll=4)
    def _(j):
        acc = jnp.zeros((_L,), jnp.int32)
        for row in range(_NS):
            acc = acc + h_ref[pl.ds(row * _SKEW + j * _L, _L)]
        rowbuf_v[pl.ds(j * _L, _L)] = acc

    pltpu.sync_copy(rowbuf_v.at[pl.ds(0, nb)], sh_ref.at[pl.ds(sid * nb, nb)])
    plsc.subcore_barrier()
    for row in range(_NS):
        pltpu.sync_copy(
            sh_ref.at[pl.ds(row * nb, nb)], h_ref.at[pl.ds(row * _ROW, nb)]
        )

    @plsc.parallel_loop(0, nb // _L, unroll=4)
    def _(j):
        acc = jnp.zeros((_L,), jnp.int32)
        for row in range(_NS):
            acc = acc + h_ref[pl.ds(row * _ROW + j * _L, _L)]
        ghist_v[pl.ds(j * _L, _L)] = acc

    return _cum_search(ghist_v, tmpa_v, tmpb_v, r_spl, nb)


def _sc_select_make():
    mesh = plsc.VectorSubcoreMesh(
        core_axis_name="c", subcore_axis_name="s", num_cores=2, num_subcores=_NS
    )

    @functools.partial(
        pl.kernel,
        out_type=jax.ShapeDtypeStruct((_L,), jnp.int32),
        mesh=mesh,
        compiler_params=pltpu.CompilerParams(needs_layout_passes=False),
        scratch_types=dict(
            p_v=pltpu.VMEM((_PER_T,), jnp.int32),
            h_v=pltpu.VMEM((_NS * _ROW,), jnp.int32),
            rowbuf_v=pltpu.VMEM((2048,), jnp.int32),
            ghist_v=pltpu.VMEM((2048,), jnp.int32),
            r_v=pltpu.VMEM((3 * _L,), jnp.int32),
            tmpa_v=pltpu.VMEM((_L,), jnp.int32),
            tmpb_v=pltpu.VMEM((_L,), jnp.int32),
            out_v=pltpu.VMEM((_L,), jnp.int32),
            sh_a=pltpu.VMEM_SHARED((_NS * 1024,), jnp.int32),
            sh_b=pltpu.VMEM_SHARED((_NS * 2048,), jnp.int32),
            sh_c=pltpu.VMEM_SHARED((_NS * 1024,), jnp.int32),
        ),
    )
    def sc_select(p_hbm, st_hbm, ans_hbm, *, p_v, h_v, rowbuf_v, ghist_v,
                  r_v, tmpa_v, tmpb_v, out_v, sh_a, sh_b, sh_c):
        cid = lax.axis_index("c")
        sid = lax.axis_index("s")
        pltpu.sync_copy(p_hbm.at[pl.ds(sid * _PER_T, _PER_T)], p_v)
        pltpu.sync_copy(st_hbm.at[pl.ds(0, 3 * _L)], r_v)
        r1 = r_v[pl.ds(0, _L)]

        b1, cb1 = _hist_round(
            p_v, h_v, rowbuf_v, sh_a, ghist_v, tmpa_v, tmpb_v, sid, r1, 1024,
            lambda v: lax.shift_right_logical(v, 21),
            lambda v: jnp.ones((_L,), jnp.bool_),
        )
        r2 = r1 - cb1

        b2, cb2 = _hist_round(
            p_v, h_v, rowbuf_v, sh_b, ghist_v, tmpa_v, tmpb_v, sid, r2, 2048,
            lambda v: lax.shift_right_logical(v, 10) & 0x7FF,
            lambda v: lax.shift_right_logical(v, 21) == b1,
        )
        r3 = r2 - cb2
        pre2 = (b1 << 11) | b2

        b3, _ = _hist_round(
            p_v, h_v, rowbuf_v, sh_c, ghist_v, tmpa_v, tmpb_v, sid, r3, 1024,
            lambda v: v & 0x3FF,
            lambda v: lax.shift_right_logical(v, 10) == pre2,
        )

        ans = (b1 << 21) | (b2 << 10) | b3

        @pl.when((cid == 0) & (sid == 0))
        def _():
            out_v[...] = ans
            pltpu.sync_copy(out_v, ans_hbm)

    return sc_select


def kernel(x):
    p, stats = pl.pallas_call(
        _tc1_body,
        out_shape=(
            jax.ShapeDtypeStruct(x.shape, jnp.int32),
            jax.ShapeDtypeStruct((8, 128), jnp.int32),
        ),
    )(x)
    st_flat = stats.reshape(-1)
    ansv = _sc_select_make()(p.reshape(-1), st_flat)
    return pl.pallas_call(
        _tc2_body,
        out_shape=jax.ShapeDtypeStruct(x.shape, x.dtype),
        in_specs=[
            pl.BlockSpec(memory_space=pltpu.VMEM),
            pl.BlockSpec(memory_space=pltpu.SMEM),
            pl.BlockSpec(memory_space=pltpu.SMEM),
        ],
        out_specs=pl.BlockSpec(memory_space=pltpu.VMEM),
    )(x, ansv, st_flat)


# speedup vs baseline: 1.2903x; 1.0913x over previous
import functools

import jax
import jax.numpy as jnp
from jax import lax
from jax.experimental import pallas as pl
from jax.experimental.pallas import tpu as pltpu
from jax.experimental.pallas import tpu_sc as plsc

_PCT = 95
_MIN_SCALE = 1e-06
_MAX_SCALE = 1000000.0
_INF_BITS = 0x7F800000

_N = 128 * 8192
_NS = 16
_L = 16
_PER_T = _N // _NS
_VECS = _PER_T // _L


def _tc1_body(x_ref, p_ref, s_ref):
    x = x_ref[:]
    a = jnp.abs(x)
    mask = a > 1e-08
    n0 = jnp.sum(mask.astype(jnp.int32))
    n0f = n0.astype(jnp.float32)
    s = jnp.sum(jnp.where(mask, a, 0.0))
    mean = s / jnp.maximum(n0f, 1.0)
    d = a - mean
    ss = jnp.sum(jnp.where(mask, d * d, 0.0))
    var = ss / jnp.maximum(n0f - 1.0, 1.0)
    std = jnp.sqrt(var)
    refined = mask & (jnp.abs(d) <= 3.0 * std)
    nr = jnp.sum(refined.astype(jnp.int32))
    use_refined = (n0 > 10) & (nr > 0)
    n = jnp.where(use_refined, nr, n0)
    k = jnp.clip((_PCT * n) // 100, 0, n - 1)
    r = k + 1
    sel = (refined & use_refined) | (mask & jnp.logical_not(use_refined))
    bits = lax.bitcast_convert_type(a, jnp.int32)
    p_ref[:] = jnp.where(sel, bits, _INF_BITS)
    cols = lax.broadcasted_iota(jnp.int32, (8, 128), 1)
    s_ref[:] = jnp.where(
        cols < 16, r, jnp.where(cols < 32, n, jnp.where(cols < 48, n0, 0))
    )


def _tc2_body(x_ref, a_ref, s_ref, o_ref):
    ans = a_ref[0]
    n = s_ref[16]
    n0 = s_ref[32]
    val = lax.bitcast_convert_type(ans, jnp.float32)
    val = jnp.where(n == 0, 1.0, val)
    value = jnp.clip(val, _MIN_SCALE, _MAX_SCALE)
    value = jnp.where(n0 == 0, 1.0, value)
    value = jnp.clip(value, _MIN_SCALE, _MAX_SCALE)
    o_ref[:] = x_ref[:] / (value + 1e-08)


def _cum_search(ghist_v, cums_ref, tmpa_v, tmpb_v, r_spl, nb):
    nchunk = nb // _L

    @plsc.parallel_loop(0, nchunk, unroll=8)
    def _(j):
        cums_ref[pl.ds(j * _L, _L)] = plsc.cumsum(ghist_v[pl.ds(j * _L, _L)])

    lane = lax.broadcasted_iota(jnp.int32, (_L,), 0)
    zero = jnp.zeros((_L,), jnp.int32)

    def group(g, carry):
        gtot, cfound, cprefix = carry
        idx = g * (_L * _L) + lane * _L + (_L - 1)
        totals = plsc.load_gather(cums_ref, [idx])
        gcs = plsc.cumsum(totals) + gtot
        ge = gcs >= r_spl
        anyv = plsc.all_reduce_population_count(ge)
        ffs = jnp.minimum(plsc.all_reduce_ffs(ge), _L - 1)
        tmpa_v[...] = gcs - totals
        pref = plsc.load_gather(tmpa_v, [ffs])
        tmpb_v[...] = gcs
        gtot_new = plsc.load_gather(tmpb_v, [jnp.full((_L,), _L - 1, jnp.int32)])
        newly = (cfound < 0) & (anyv > 0)
        cfound = jnp.where(newly, g * _L + ffs, cfound)
        cprefix = jnp.where(newly, pref, cprefix)
        return (gtot_new, cfound, cprefix)

    init = (zero, zero - 1, zero)
    _, cfound, cprefix = lax.fori_loop(0, nchunk // _L, group, init)
    c = jnp.max(jnp.maximum(cfound, 0))
    cs = cums_ref[pl.ds(c * _L, _L)] + cprefix
    h = ghist_v[pl.ds(c * _L, _L)]
    ge = cs >= r_spl
    ffs = jnp.minimum(plsc.all_reduce_ffs(ge), _L - 1)
    tmpa_v[...] = cs - h
    cbel = plsc.load_gather(tmpa_v, [ffs])
    return c * _L + ffs, cbel


_SKEW = 2081
_ROW = 2080
_DUMMY = 2064


def _zero_hist(h_ref, nb):
    zero = jnp.zeros((_L,), jnp.int32)
    for row in range(_NS):

        @plsc.parallel_loop(0, nb // _L, unroll=8)
        def _(col, row=row):
            h_ref[pl.ds(row * _SKEW + col * _L, _L)] = zero


def _hist_round(p_v, h_ref, rowbuf_v, sh_ref, shg_ref, ghist_v, tmpa_v,
                tmpb_v, sid, r_spl, nb, bin_fn, mask_fn):
    _zero_hist(h_ref, nb)
    lane_off = lax.broadcasted_iota(jnp.int32, (_L,), 0) * _SKEW
    ones = jnp.ones((_L,), jnp.int32)

    @plsc.parallel_loop(0, _VECS, unroll=16)
    def _(i):
        v = p_v[pl.ds(i * _L, _L)]
        bins = jnp.where(mask_fn(v), bin_fn(v), _DUMMY)
        plsc.addupdate_scatter(h_ref, [lane_off + bins], ones)

    @plsc.parallel_loop(0, nb // _L, unroll=4)
    def _(j):
        acc = jnp.zeros((_L,), jnp.int32)
        for row in range(_NS):
            acc = acc + h_ref[pl.ds(row * _SKEW + j * _L, _L)]
        rowbuf_v[pl.ds(j * _L, _L)] = acc

    pltpu.sync_copy(rowbuf_v.at[pl.ds(0, nb)], sh_ref.at[pl.ds(sid * nb, nb)])
    plsc.subcore_barrier()

    sb = nb // _NS
    for row in range(_NS):
        pltpu.sync_copy(
            sh_ref.at[pl.ds(row * nb + sid * sb, sb)],
            h_ref.at[pl.ds(row * _ROW, sb)],
        )

    @plsc.parallel_loop(0, sb // _L, unroll=4)
    def _(j):
        acc = jnp.zeros((_L,), jnp.int32)
        for row in range(_NS):
            acc = acc + h_ref[pl.ds(row * _ROW + j * _L, _L)]
        rowbuf_v[pl.ds(j * _L, _L)] = acc

    pltpu.sync_copy(rowbuf_v.at[pl.ds(0, sb)], shg_ref.at[pl.ds(sid * sb, sb)])
    plsc.subcore_barrier()
    pltpu.sync_copy(shg_ref.at[pl.ds(0, nb)], ghist_v.at[pl.ds(0, nb)])

    return _cum_search(ghist_v, h_ref, tmpa_v, tmpb_v, r_spl, nb)


def _sc_select_make():
    mesh = plsc.VectorSubcoreMesh(
        core_axis_name="c", subcore_axis_name="s", num_cores=2, num_subcores=_NS
    )

    @functools.partial(
        pl.kernel,
        out_type=jax.ShapeDtypeStruct((_L,), jnp.int32),
        mesh=mesh,
        compiler_params=pltpu.CompilerParams(needs_layout_passes=False),
        scratch_types=dict(
            p_v=pltpu.VMEM((_PER_T,), jnp.int32),
            h_v=pltpu.VMEM((_NS * _ROW,), jnp.int32),
            rowbuf_v=pltpu.VMEM((2048,), jnp.int32),
            ghist_v=pltpu.VMEM((2048,), jnp.int32),
            r_v=pltpu.VMEM((3 * _L,), jnp.int32),
            tmpa_v=pltpu.VMEM((_L,), jnp.int32),
            tmpb_v=pltpu.VMEM((_L,), jnp.int32),
            out_v=pltpu.VMEM((_L,), jnp.int32),
            sh_a=pltpu.VMEM_SHARED((_NS * 1024,), jnp.int32),
            sh_g=pltpu.VMEM_SHARED((2048,), jnp.int32),
            sh_b=pltpu.VMEM_SHARED((_NS * 2048,), jnp.int32),
            sh_c=pltpu.VMEM_SHARED((_NS * 1024,), jnp.int32),
        ),
    )
    def sc_select(p_hbm, st_hbm, ans_hbm, *, p_v, h_v, rowbuf_v, ghist_v,
                  r_v, tmpa_v, tmpb_v, out_v, sh_a, sh_b, sh_c, sh_g):
        cid = lax.axis_index("c")
        sid = lax.axis_index("s")
        pltpu.sync_copy(p_hbm.at[pl.ds(sid * _PER_T, _PER_T)], p_v)
        pltpu.sync_copy(st_hbm.at[pl.ds(0, 3 * _L)], r_v)
        r1 = r_v[pl.ds(0, _L)]

        b1, cb1 = _hist_round(
            p_v, h_v, rowbuf_v, sh_a, sh_g, ghist_v, tmpa_v, tmpb_v, sid,
            r1, 1024,
            lambda v: lax.shift_right_logical(v, 21),
            lambda v: jnp.ones((_L,), jnp.bool_),
        )
        r2 = r1 - cb1

        b2, cb2 = _hist_round(
            p_v, h_v, rowbuf_v, sh_b, sh_g, ghist_v, tmpa_v, tmpb_v, sid,
            r2, 2048,
            lambda v: lax.shift_right_logical(v, 10) & 0x7FF,
            lambda v: lax.shift_right_logical(v, 21) == b1,
        )
        r3 = r2 - cb2
        pre2 = (b1 << 11) | b2

        b3, _ = _hist_round(
            p_v, h_v, rowbuf_v, sh_c, sh_g, ghist_v, tmpa_v, tmpb_v, sid,
            r3, 1024,
            lambda v: v & 0x3FF,
            lambda v: lax.shift_right_logical(v, 10) == pre2,
        )

        ans = (b1 << 21) | (b2 << 10) | b3

        @pl.when((cid == 0) & (sid == 0))
        def _():
            out_v[...] = ans
            pltpu.sync_copy(out_v, ans_hbm)

    return sc_select


def kernel(x):
    p, stats = pl.pallas_call(
        _tc1_body,
        out_shape=(
            jax.ShapeDtypeStruct(x.shape, jnp.int32),
            jax.ShapeDtypeStruct((8, 128), jnp.int32),
        ),
    )(x)
    st_flat = stats.reshape(-1)
    ansv = _sc_select_make()(p.reshape(-1), st_flat)
    return pl.pallas_call(
        _tc2_body,
        out_shape=jax.ShapeDtypeStruct(x.shape, x.dtype),
        in_specs=[
            pl.BlockSpec(memory_space=pltpu.VMEM),
            pl.BlockSpec(memory_space=pltpu.SMEM),
            pl.BlockSpec(memory_space=pltpu.SMEM),
        ],
        out_specs=pl.BlockSpec(memory_space=pltpu.VMEM),
    )(x, ansv, st_flat)
